# Initial kernel scaffold; baseline (speedup 1.0000x reference)
import jax
import jax.numpy as jnp
from jax.experimental import pallas as pl


def kernel(x, edge_index, W1, b1, g1, be1, m1, v1, W2, b2, g2, be2, m2, v2, Wl, bl):
    raise NotImplementedError("write your pallas kernel here")


# speedup vs baseline: 12.7684x; 12.7684x over previous
import functools

import jax
import jax.numpy as jnp
from jax import lax
from jax.experimental import pallas as pl
from jax.experimental.pallas import tpu as pltpu
from jax.experimental.pallas import tpu_sc as plsc

N = 10000
E = 320000
D_IN = 128
H = 128
H2 = 64

NC = 2
NS = 16
NW = NC * NS
EPW = E // NW
B = 125
NB = EPW // B

HALF = 5120
JUNK = 512
ACC1 = HALF + JUNK
ZPT = ACC1 // NS
CPT = HALF // NS
N_OUT = 2 * HALF

ACC2 = HALF

_MESH = plsc.VectorSubcoreMesh(core_axis_name="c", subcore_axis_name="s")


HC = CPT // 2


@functools.partial(
    pl.kernel,
    out_type=jax.ShapeDtypeStruct((NC * N_OUT, H), jnp.float32),
    mesh=_MESH,
    scratch_types=[
        pltpu.VMEM((NB, B), jnp.int32),
        pltpu.VMEM((B, H), jnp.float32),
        pltpu.VMEM((ZPT // 2, H), jnp.float32),
        pltpu.VMEM_SHARED((ACC1, H), jnp.float32),
    ],
)
def _deg_sc(ones_hbm, dst_hbm, zeros_hbm, out_hbm, didx, rows, zbuf, acc):
    c = lax.axis_index("c")
    s = lax.axis_index("s")
    wid = c * NS + s
    pltpu.sync_copy(ones_hbm, rows)
    for p in range(2):
        if p > 0:
            plsc.subcore_barrier()
        pltpu.sync_copy(zeros_hbm, zbuf)
        for j in range(2):
            pltpu.sync_copy(
                zbuf, acc.at[pl.ds(s * ZPT + j * (ZPT // 2), ZPT // 2)])
        pltpu.sync_copy(dst_hbm.at[p * NW + wid], didx)
        plsc.subcore_barrier()

        def blk(k, carry):
            pltpu.sync_copy(rows, acc.at[didx.at[k]], add=True)
            return carry

        lax.fori_loop(0, NB, blk, 0)
        plsc.subcore_barrier()
        for j in range(2):
            pltpu.sync_copy(acc.at[pl.ds(s * CPT + j * HC, HC)],
                            zbuf.at[pl.ds(0, HC)])
            pltpu.sync_copy(
                zbuf.at[pl.ds(0, HC)],
                out_hbm.at[pl.ds(c * N_OUT + p * HALF + s * CPT + j * HC, HC)])


@functools.partial(
    pl.kernel,
    out_type=jax.ShapeDtypeStruct((NC * N_OUT, H), jnp.float32),
    mesh=_MESH,
    scratch_types=[
        pltpu.VMEM((NB, B), jnp.int32),
        pltpu.VMEM((NB, B), jnp.int32),
        pltpu.VMEM((B, H), jnp.float32),
        pltpu.VMEM((ZPT // 2, H), jnp.float32),
        pltpu.VMEM_SHARED((ACC1, H), jnp.float32),
        pltpu.SemaphoreType.DMA,
    ],
)
def _prop1(h_hbm, src_hbm, dst_hbm, zeros_hbm, out_hbm,
           sidx, didx, rows, zbuf, acc, sem):
    c = lax.axis_index("c")
    s = lax.axis_index("s")
    wid = c * NS + s
    pltpu.sync_copy(src_hbm.at[wid], sidx)
    for p in range(2):
        if p > 0:
            plsc.subcore_barrier()
        pltpu.sync_copy(zeros_hbm, zbuf)
        for j in range(2):
            pltpu.sync_copy(
                zbuf, acc.at[pl.ds(s * ZPT + j * (ZPT // 2), ZPT // 2)])
        pltpu.sync_copy(dst_hbm.at[p * NW + wid], didx)
        plsc.subcore_barrier()

        def blk(k, carry):
            pltpu.async_copy(h_hbm.at[sidx.at[k]], rows, sem).wait()
            pltpu.sync_copy(rows, acc.at[didx.at[k]], add=True)
            return carry

        lax.fori_loop(0, NB, blk, 0)
        plsc.subcore_barrier()
        for j in range(2):
            pltpu.sync_copy(acc.at[pl.ds(s * CPT + j * HC, HC)],
                            zbuf.at[pl.ds(0, HC)])
            pltpu.sync_copy(
                zbuf.at[pl.ds(0, HC)],
                out_hbm.at[pl.ds(c * N_OUT + p * HALF + s * CPT + j * HC, HC)])


@functools.partial(
    pl.kernel,
    out_type=jax.ShapeDtypeStruct((NC * ACC2, H), jnp.float32),
    mesh=_MESH,
    scratch_types=[
        pltpu.VMEM((NB, B), jnp.int32),
        pltpu.VMEM((NB, B), jnp.int32),
        pltpu.VMEM((B, H), jnp.float32),
        pltpu.VMEM((ZPT // 2, H), jnp.float32),
        pltpu.VMEM_SHARED((ACC2, H), jnp.float32),
        pltpu.SemaphoreType.DMA,
    ],
)
def _prop2(h_hbm, src_hbm, dst_hbm, zeros_hbm, out_hbm,
           sidx, didx, rows, zbuf, acc, sem):
    c = lax.axis_index("c")
    s = lax.axis_index("s")
    wid = c * NS + s
    pltpu.sync_copy(src_hbm.at[wid], sidx)
    pltpu.sync_copy(dst_hbm.at[wid], didx)
    pltpu.sync_copy(zeros_hbm, zbuf)
    for j in range(2):
        pltpu.sync_copy(zbuf.at[pl.ds(0, HC)],
                        acc.at[pl.ds(s * CPT + j * HC, HC)])
    plsc.subcore_barrier()

    def blk(k, carry):
        pltpu.async_copy(h_hbm.at[sidx.at[k]], rows, sem).wait()
        pltpu.sync_copy(rows, acc.at[didx.at[k]], add=True)
        return carry

    lax.fori_loop(0, NB, blk, 0)
    plsc.subcore_barrier()
    for j in range(2):
        pltpu.sync_copy(acc.at[pl.ds(s * CPT + j * HC, HC)],
                        zbuf.at[pl.ds(0, HC)])
        pltpu.sync_copy(
            zbuf.at[pl.ds(0, HC)],
            out_hbm.at[pl.ds(c * ACC2 + s * CPT + j * HC, HC)])


_R = 1024


def _k2_body(x_ref, w_ref, dp_ref, h1s_ref, dinv_ref):
    deg = dp_ref[0, :, 0:1] + dp_ref[1, :, 0:1] + 1.0
    dinv = lax.rsqrt(deg)
    h1 = jnp.dot(x_ref[...], w_ref[...], preferred_element_type=jnp.float32)
    h1s_ref[...] = h1 * dinv
    dinv_ref[...] = dinv


_k2 = pl.pallas_call(
    _k2_body,
    grid=(pl.cdiv(N, _R),),
    in_specs=[
        pl.BlockSpec((_R, D_IN), lambda i: (i, 0)),
        pl.BlockSpec((D_IN, H), lambda i: (0, 0)),
        pl.BlockSpec((2, _R, H), lambda i: (0, i, 0)),
    ],
    out_specs=[
        pl.BlockSpec((_R, H), lambda i: (i, 0)),
        pl.BlockSpec((_R, 1), lambda i: (i, 0)),
    ],
    out_shape=[
        jax.ShapeDtypeStruct((N, H), jnp.float32),
        jax.ShapeDtypeStruct((N, 1), jnp.float32),
    ],
)


def _k4_body(p_ref, h1s_ref, dinv_ref, a_ref, d_ref, w2_ref, t2_ref, h2s_ref):
    dinv = dinv_ref[...]
    ssum = p_ref[0] + p_ref[1] + h1s_ref[...]
    z = jnp.maximum(dinv * ssum * a_ref[...] + d_ref[...], 0.0)
    h2 = jnp.dot(z, w2_ref[...], preferred_element_type=jnp.float32) * dinv
    zero = jnp.zeros_like(h2)
    h2s_ref[...] = h2
    t2_ref[:, 0, :H2] = h2
    t2_ref[:, 0, H2:] = zero
    t2_ref[:, 1, :H2] = zero
    t2_ref[:, 1, H2:] = h2


_k4 = pl.pallas_call(
    _k4_body,
    grid=(pl.cdiv(N, _R),),
    in_specs=[
        pl.BlockSpec((2, _R, H), lambda i: (0, i, 0)),
        pl.BlockSpec((_R, H), lambda i: (i, 0)),
        pl.BlockSpec((_R, 1), lambda i: (i, 0)),
        pl.BlockSpec((1, H), lambda i: (0, 0)),
        pl.BlockSpec((1, H), lambda i: (0, 0)),
        pl.BlockSpec((H, H2), lambda i: (0, 0)),
    ],
    out_specs=[
        pl.BlockSpec((_R, 2, H), lambda i: (i, 0, 0)),
        pl.BlockSpec((_R, H2), lambda i: (i, 0)),
    ],
    out_shape=[
        jax.ShapeDtypeStruct((N, 2, H), jnp.float32),
        jax.ShapeDtypeStruct((N, H2), jnp.float32),
    ],
)


def _k6_body(p_ref, h2s_ref, dinv_ref, a_ref, d_ref, wl_ref, bl_ref, out_ref):
    dinv = dinv_ref[...]
    ssum = p_ref[0] + p_ref[1] + h2s_ref[...]
    z = jnp.maximum(dinv * ssum * a_ref[...] + d_ref[...], 0.0)
    out_ref[...] = jnp.sum(z * wl_ref[...], axis=1, keepdims=True) + bl_ref[...]


_k6 = pl.pallas_call(
    _k6_body,
    grid=(pl.cdiv(N, _R),),
    in_specs=[
        pl.BlockSpec((2, _R, H2), lambda i: (0, i, 0)),
        pl.BlockSpec((_R, H2), lambda i: (i, 0)),
        pl.BlockSpec((_R, 1), lambda i: (i, 0)),
        pl.BlockSpec((1, H2), lambda i: (0, 0)),
        pl.BlockSpec((1, H2), lambda i: (0, 0)),
        pl.BlockSpec((1, H2), lambda i: (0, 0)),
        pl.BlockSpec((1, 1), lambda i: (0, 0)),
    ],
    out_specs=pl.BlockSpec((_R, 1), lambda i: (i, 0)),
    out_shape=jax.ShapeDtypeStruct((N, 1), jnp.float32),
)


def kernel(x, edge_index, W1, b1, g1, be1, m1, v1, W2, b2, g2, be2, m2, v2,
           Wl, bl):
    ei = edge_index.astype(jnp.int32)
    src = ei[0]
    dst = ei[1]

    src3d = src.reshape(NW, NB, B)
    junk = HALF + (dst & (JUNK - 1))
    dst_lo = jnp.where(dst < HALF, dst, junk)
    dst_hi = jnp.where(dst >= HALF, dst - HALF, junk)
    dst_lh = jnp.stack([dst_lo, dst_hi]).reshape(2 * NW, NB, B)
    src_pk = (2 * src + (dst & 1)).reshape(NW, NB, B)
    dst_pk = (dst >> 1).reshape(NW, NB, B)

    ones128 = jnp.ones((B, H), jnp.float32)
    zeros128 = jnp.zeros((ZPT // 2, H), jnp.float32)

    degp = _deg_sc(ones128, dst_lh, zeros128).reshape(NC, N_OUT, H)[:, :N]
    h1s, dinv = _k2(x, W1, degp)

    p1 = _prop1(h1s, src3d, dst_lh, zeros128).reshape(NC, N_OUT, H)[:, :N]

    eps = 1e-5
    a1 = g1 * lax.rsqrt(v1 + eps)
    d1 = (b1 - m1) * a1 + be1
    t2, h2s = _k4(p1, h1s, dinv, a1.reshape(1, H), d1.reshape(1, H), W2)

    p2 = _prop2(t2.reshape(2 * N, H), src_pk, dst_pk, zeros128)
    p2 = p2.reshape(NC, ACC2, 2, H2).reshape(NC, N_OUT, H2)[:, :N]

    a2 = g2 * lax.rsqrt(v2 + eps)
    d2 = (b2 - m2) * a2 + be2
    logits = _k6(p2, h2s, dinv, a2.reshape(1, H2), d2.reshape(1, H2),
                 Wl.reshape(1, H2), bl.reshape(1, 1))
    return logits[:, 0]


# speedup vs baseline: 15.5503x; 1.2179x over previous
import functools

import jax
import jax.numpy as jnp
from jax import lax
from jax.experimental import pallas as pl
from jax.experimental.pallas import tpu as pltpu
from jax.experimental.pallas import tpu_sc as plsc

N = 10000
E = 320000
D_IN = 128
H = 128
H2 = 64

NC = 2
NS = 16
NW = NC * NS
EPW = E // NW
B = 125
NB = EPW // B

HALF = 5120
JUNK = 512
ACC1 = HALF + JUNK
ZPT = ACC1 // NS
CPT = HALF // NS
N_OUT = 2 * HALF

ACC2 = HALF

_MESH = plsc.VectorSubcoreMesh(core_axis_name="c", subcore_axis_name="s")


HC = CPT // 2


@functools.partial(
    pl.kernel,
    out_type=jax.ShapeDtypeStruct((NC * N_OUT, H), jnp.float32),
    mesh=_MESH,
    scratch_types=[
        pltpu.VMEM((NB, B), jnp.int32),
        pltpu.VMEM((B, H), jnp.float32),
        pltpu.VMEM((ZPT // 2, H), jnp.float32),
        pltpu.VMEM_SHARED((ACC1, H), jnp.float32),
    ],
)
def _deg_sc(ones_hbm, dst_hbm, zeros_hbm, out_hbm, didx, rows, zbuf, acc):
    c = lax.axis_index("c")
    s = lax.axis_index("s")
    wid = c * NS + s
    pltpu.sync_copy(ones_hbm, rows)
    for p in range(2):
        if p > 0:
            plsc.subcore_barrier()
        pltpu.sync_copy(zeros_hbm, zbuf)
        for j in range(2):
            pltpu.sync_copy(
                zbuf, acc.at[pl.ds(s * ZPT + j * (ZPT // 2), ZPT // 2)])
        pltpu.sync_copy(dst_hbm.at[p * NW + wid], didx)
        plsc.subcore_barrier()

        def blk(k, carry):
            pltpu.sync_copy(rows, acc.at[didx.at[k]], add=True)
            return carry

        lax.fori_loop(0, NB, blk, 0)
        plsc.subcore_barrier()
        for j in range(2):
            pltpu.sync_copy(acc.at[pl.ds(s * CPT + j * HC, HC)],
                            zbuf.at[pl.ds(0, HC)])
            pltpu.sync_copy(
                zbuf.at[pl.ds(0, HC)],
                out_hbm.at[pl.ds(c * N_OUT + p * HALF + s * CPT + j * HC, HC)])


@functools.partial(
    pl.kernel,
    out_type=jax.ShapeDtypeStruct((NC * N_OUT, H), jnp.float32),
    mesh=_MESH,
    scratch_types=[
        pltpu.VMEM((NB, B), jnp.int32),
        pltpu.VMEM((NB, B), jnp.int32),
        pltpu.VMEM((2, B, H), jnp.float32),
        pltpu.VMEM((ZPT // 2, H), jnp.float32),
        pltpu.VMEM_SHARED((ACC1, H), jnp.float32),
        pltpu.SemaphoreType.DMA,
    ],
)
def _prop1(h_hbm, src_hbm, dst_hbm, zeros_hbm, out_hbm,
           sidx, didx, rows, zbuf, acc, sem):
    c = lax.axis_index("c")
    s = lax.axis_index("s")
    wid = c * NS + s
    pltpu.sync_copy(src_hbm.at[wid], sidx)
    for p in range(2):
        if p > 0:
            plsc.subcore_barrier()
        pltpu.sync_copy(zeros_hbm, zbuf)
        for j in range(2):
            pltpu.sync_copy(
                zbuf, acc.at[pl.ds(s * ZPT + j * (ZPT // 2), ZPT // 2)])
        pltpu.sync_copy(dst_hbm.at[p * NW + wid], didx)
        plsc.subcore_barrier()
        pltpu.async_copy(h_hbm.at[sidx.at[0]], rows.at[0], sem)

        def blk(k, carry):
            buf = lax.rem(k, 2)
            pltpu.make_async_copy(h_hbm.at[sidx.at[k]], rows.at[buf],
                                  sem).wait()

            @pl.when(k + 1 < NB)
            def _():
                pltpu.async_copy(h_hbm.at[sidx.at[k + 1]],
                                 rows.at[1 - buf], sem)

            pltpu.sync_copy(rows.at[buf], acc.at[didx.at[k]], add=True)
            return carry

        lax.fori_loop(0, NB, blk, 0)
        plsc.subcore_barrier()
        for j in range(2):
            pltpu.sync_copy(acc.at[pl.ds(s * CPT + j * HC, HC)],
                            zbuf.at[pl.ds(0, HC)])
            pltpu.sync_copy(
                zbuf.at[pl.ds(0, HC)],
                out_hbm.at[pl.ds(c * N_OUT + p * HALF + s * CPT + j * HC, HC)])


@functools.partial(
    pl.kernel,
    out_type=jax.ShapeDtypeStruct((NC * ACC2, H), jnp.float32),
    mesh=_MESH,
    scratch_types=[
        pltpu.VMEM((NB, B), jnp.int32),
        pltpu.VMEM((NB, B), jnp.int32),
        pltpu.VMEM((2, B, H), jnp.float32),
        pltpu.VMEM((ZPT // 2, H), jnp.float32),
        pltpu.VMEM_SHARED((ACC2, H), jnp.float32),
        pltpu.SemaphoreType.DMA,
    ],
)
def _prop2(h_hbm, src_hbm, dst_hbm, zeros_hbm, out_hbm,
           sidx, didx, rows, zbuf, acc, sem):
    c = lax.axis_index("c")
    s = lax.axis_index("s")
    wid = c * NS + s
    pltpu.sync_copy(src_hbm.at[wid], sidx)
    pltpu.sync_copy(dst_hbm.at[wid], didx)
    pltpu.sync_copy(zeros_hbm, zbuf)
    for j in range(2):
        pltpu.sync_copy(zbuf.at[pl.ds(0, HC)],
                        acc.at[pl.ds(s * CPT + j * HC, HC)])
    plsc.subcore_barrier()
    pltpu.async_copy(h_hbm.at[sidx.at[0]], rows.at[0], sem)

    def blk(k, carry):
        buf = lax.rem(k, 2)
        pltpu.make_async_copy(h_hbm.at[sidx.at[k]], rows.at[buf],
                              sem).wait()

        @pl.when(k + 1 < NB)
        def _():
            pltpu.async_copy(h_hbm.at[sidx.at[k + 1]], rows.at[1 - buf], sem)

        pltpu.sync_copy(rows.at[buf], acc.at[didx.at[k]], add=True)
        return carry

    lax.fori_loop(0, NB, blk, 0)
    plsc.subcore_barrier()
    for j in range(2):
        pltpu.sync_copy(acc.at[pl.ds(s * CPT + j * HC, HC)],
                        zbuf.at[pl.ds(0, HC)])
        pltpu.sync_copy(
            zbuf.at[pl.ds(0, HC)],
            out_hbm.at[pl.ds(c * ACC2 + s * CPT + j * HC, HC)])


_R = 1024


def _k2_body(x_ref, w_ref, dp_ref, h1s_ref, dinv_ref):
    deg = dp_ref[0, :, 0:1] + dp_ref[1, :, 0:1] + 1.0
    dinv = lax.rsqrt(deg)
    h1 = jnp.dot(x_ref[...], w_ref[...], preferred_element_type=jnp.float32)
    h1s_ref[...] = h1 * dinv
    dinv_ref[...] = dinv


_k2 = pl.pallas_call(
    _k2_body,
    grid=(pl.cdiv(N, _R),),
    in_specs=[
        pl.BlockSpec((_R, D_IN), lambda i: (i, 0)),
        pl.BlockSpec((D_IN, H), lambda i: (0, 0)),
        pl.BlockSpec((2, _R, H), lambda i: (0, i, 0)),
    ],
    out_specs=[
        pl.BlockSpec((_R, H), lambda i: (i, 0)),
        pl.BlockSpec((_R, 1), lambda i: (i, 0)),
    ],
    out_shape=[
        jax.ShapeDtypeStruct((N, H), jnp.float32),
        jax.ShapeDtypeStruct((N, 1), jnp.float32),
    ],
)


def _k4_body(p_ref, h1s_ref, dinv_ref, a_ref, d_ref, w2_ref, t2_ref, h2s_ref):
    dinv = dinv_ref[...]
    ssum = p_ref[0] + p_ref[1] + h1s_ref[...]
    z = jnp.maximum(dinv * ssum * a_ref[...] + d_ref[...], 0.0)
    h2 = jnp.dot(z, w2_ref[...], preferred_element_type=jnp.float32) * dinv
    zero = jnp.zeros_like(h2)
    h2s_ref[...] = h2
    t2_ref[:, 0, :H2] = h2
    t2_ref[:, 0, H2:] = zero
    t2_ref[:, 1, :H2] = zero
    t2_ref[:, 1, H2:] = h2


_k4 = pl.pallas_call(
    _k4_body,
    grid=(pl.cdiv(N, _R),),
    in_specs=[
        pl.BlockSpec((2, _R, H), lambda i: (0, i, 0)),
        pl.BlockSpec((_R, H), lambda i: (i, 0)),
        pl.BlockSpec((_R, 1), lambda i: (i, 0)),
        pl.BlockSpec((1, H), lambda i: (0, 0)),
        pl.BlockSpec((1, H), lambda i: (0, 0)),
        pl.BlockSpec((H, H2), lambda i: (0, 0)),
    ],
    out_specs=[
        pl.BlockSpec((_R, 2, H), lambda i: (i, 0, 0)),
        pl.BlockSpec((_R, H2), lambda i: (i, 0)),
    ],
    out_shape=[
        jax.ShapeDtypeStruct((N, 2, H), jnp.float32),
        jax.ShapeDtypeStruct((N, H2), jnp.float32),
    ],
)


def _k6_body(p_ref, h2s_ref, dinv_ref, a_ref, d_ref, wl_ref, bl_ref, out_ref):
    dinv = dinv_ref[...]
    ssum = p_ref[0] + p_ref[1] + h2s_ref[...]
    z = jnp.maximum(dinv * ssum * a_ref[...] + d_ref[...], 0.0)
    out_ref[...] = jnp.sum(z * wl_ref[...], axis=1, keepdims=True) + bl_ref[...]


_k6 = pl.pallas_call(
    _k6_body,
    grid=(pl.cdiv(N, _R),),
    in_specs=[
        pl.BlockSpec((2, _R, H2), lambda i: (0, i, 0)),
        pl.BlockSpec((_R, H2), lambda i: (i, 0)),
        pl.BlockSpec((_R, 1), lambda i: (i, 0)),
        pl.BlockSpec((1, H2), lambda i: (0, 0)),
        pl.BlockSpec((1, H2), lambda i: (0, 0)),
        pl.BlockSpec((1, H2), lambda i: (0, 0)),
        pl.BlockSpec((1, 1), lambda i: (0, 0)),
    ],
    out_specs=pl.BlockSpec((_R, 1), lambda i: (i, 0)),
    out_shape=jax.ShapeDtypeStruct((N, 1), jnp.float32),
)


def kernel(x, edge_index, W1, b1, g1, be1, m1, v1, W2, b2, g2, be2, m2, v2,
           Wl, bl):
    ei = edge_index.astype(jnp.int32)
    src = ei[0]
    dst = ei[1]

    src3d = src.reshape(NW, NB, B)
    junk = HALF + (dst & (JUNK - 1))
    dst_lo = jnp.where(dst < HALF, dst, junk)
    dst_hi = jnp.where(dst >= HALF, dst - HALF, junk)
    dst_lh = jnp.stack([dst_lo, dst_hi]).reshape(2 * NW, NB, B)
    src_pk = (2 * src + (dst & 1)).reshape(NW, NB, B)
    dst_pk = (dst >> 1).reshape(NW, NB, B)

    ones128 = jnp.ones((B, H), jnp.float32)
    zeros128 = jnp.zeros((ZPT // 2, H), jnp.float32)

    degp = _deg_sc(ones128, dst_lh, zeros128).reshape(NC, N_OUT, H)[:, :N]
    h1s, dinv = _k2(x, W1, degp)

    p1 = _prop1(h1s, src3d, dst_lh, zeros128).reshape(NC, N_OUT, H)[:, :N]

    eps = 1e-5
    a1 = g1 * lax.rsqrt(v1 + eps)
    d1 = (b1 - m1) * a1 + be1
    t2, h2s = _k4(p1, h1s, dinv, a1.reshape(1, H), d1.reshape(1, H), W2)

    p2 = _prop2(t2.reshape(2 * N, H), src_pk, dst_pk, zeros128)
    p2 = p2.reshape(NC, ACC2, 2, H2).reshape(NC, N_OUT, H2)[:, :N]

    a2 = g2 * lax.rsqrt(v2 + eps)
    d2 = (b2 - m2) * a2 + be2
    logits = _k6(p2, h2s, dinv, a2.reshape(1, H2), d2.reshape(1, H2),
                 Wl.reshape(1, H2), bl.reshape(1, 1))
    return logits[:, 0]
